# Initial kernel scaffold; baseline (speedup 1.0000x reference)
import jax
import jax.numpy as jnp
from jax.experimental import pallas as pl


def kernel(x, Wr, Wg, Wu, Wd, Sg, Su, Sd, expert_bias):
    raise NotImplementedError("write your pallas kernel here")


# speedup vs baseline: 1.2582x; 1.2582x over previous
import functools

import jax
import jax.numpy as jnp
from jax.experimental import pallas as pl
from jax.experimental.pallas import tpu as pltpu


def _router_weight(xb, wr_ref, bias_ref, e):
    logits = jnp.dot(xb, wr_ref[...].T, preferred_element_type=jnp.float32)
    logits = logits + bias_ref[...][None, :]
    scores = jax.nn.sigmoid(logits)
    m1 = jnp.max(scores, axis=-1)
    i1 = jnp.argmax(scores, axis=-1)
    neg = jnp.full_like(scores, -jnp.inf)
    cols = jax.lax.broadcasted_iota(jnp.int32, scores.shape, 1)
    masked = jnp.where(cols == i1[:, None], neg, scores)
    m2 = jnp.max(masked, axis=-1)
    i2 = jnp.argmax(masked, axis=-1)
    denom = m1 + m2 + 1e-6
    w1 = m1 / denom
    w2 = m2 / denom
    we = jnp.where(i1 == e, w1, jnp.where(i2 == e, w2, 0.0))
    return we


def _fused_body(x_ref, wr_ref, bias_ref, wg_ref, wu_ref, wd_ref, out_ref):
    e = pl.program_id(1)
    num_e = pl.num_programs(1)
    xb = x_ref[...]

    g = jnp.dot(xb, wg_ref[0].T, preferred_element_type=jnp.float32)
    u = jnp.dot(xb, wu_ref[0].T, preferred_element_type=jnp.float32)
    h = (g * jax.nn.sigmoid(g)) * u
    y = jnp.dot(h, wd_ref[0].T, preferred_element_type=jnp.float32)

    is_shared = e == num_e - 1
    we = jnp.where(
        is_shared,
        jnp.ones((xb.shape[0],), jnp.float32),
        _router_weight(xb, wr_ref, bias_ref, e),
    )
    contrib = we[:, None] * y

    @pl.when(e == 0)
    def _init():
        out_ref[...] = contrib

    @pl.when(e != 0)
    def _acc():
        out_ref[...] = out_ref[...] + contrib


def kernel(x, Wr, Wg, Wu, Wd, Sg, Su, Sd, expert_bias):
    bsz, seqlen, dim = x.shape
    T = bsz * seqlen
    E, hid, _ = Wg.shape
    x_flat = x.reshape(T, dim)

    wg_all = jnp.concatenate([Wg, Sg[None]], axis=0)
    wu_all = jnp.concatenate([Wu, Su[None]], axis=0)
    wd_all = jnp.concatenate([Wd, Sd[None]], axis=0)

    BM = min(1024, T)
    nt = T // BM

    out = pl.pallas_call(
        _fused_body,
        grid=(nt, E + 1),
        in_specs=[
            pl.BlockSpec((BM, dim), lambda tb, e: (tb, 0)),
            pl.BlockSpec((E, dim), lambda tb, e: (0, 0)),
            pl.BlockSpec((E,), lambda tb, e: (0,)),
            pl.BlockSpec((1, hid, dim), lambda tb, e: (e, 0, 0)),
            pl.BlockSpec((1, hid, dim), lambda tb, e: (e, 0, 0)),
            pl.BlockSpec((1, dim, hid), lambda tb, e: (e, 0, 0)),
        ],
        out_specs=pl.BlockSpec((BM, dim), lambda tb, e: (tb, 0)),
        out_shape=jax.ShapeDtypeStruct((T, dim), jnp.float32),
    )(x_flat, Wr, expert_bias, wg_all, wu_all, wd_all)

    return out.reshape(bsz, seqlen, dim)
